# baseline (speedup 1.0000x reference)
import functools

import jax
import jax.numpy as jnp
from jax import lax
from jax.experimental import pallas as pl
from jax.experimental.pallas import tpu as pltpu
from jax.experimental.pallas import tpu_sc as plsc

_P = 5
_PR = _P // 2

_B, _C, _H, _W = 2, 96, 64, 128
_NC, _NS, _L = 2, 16, 16
_NW = _NC * _NS
_TOTAL = _B * _C * _H * _W
_CHUNK = _TOTAL // _NW
_ROW_VREGS = _W // _L
_IMG = _H * _W


def _sc_body(x_hbm, out_hbm, buf):
    wid = lax.axis_index("s") * _NC + lax.axis_index("c")
    base = wid * _CHUNK
    pltpu.sync_copy(x_hbm.at[pl.ds(base, _CHUNK)], buf)

    def step(i, carry):
        l = (i % (_IMG // _L)) // _ROW_VREGS
        v = buf[pl.ds(i * _L, _L)]
        acc = jnp.zeros((_L,), jnp.float32)
        nvalid = jnp.float32(0)
        for off in range(-_PR, _PR + 1):
            m = l + off
            ok = jnp.logical_and(m >= _PR, m <= _H - 1 - _PR)
            okf = ok.astype(jnp.float32)
            acc = acc + v * okf
            nvalid = nvalid + okf
        buf[pl.ds(i * _L, _L)] = acc * _P / (nvalid * _P)
        return carry

    lax.fori_loop(0, _CHUNK // _L, step, 0, unroll=8)
    pltpu.sync_copy(buf, out_hbm.at[pl.ds(base, _CHUNK)])


@functools.partial(jax.jit)
def _sc_kernel(xflat):
    run = pl.kernel(
        _sc_body,
        out_type=jax.ShapeDtypeStruct((_TOTAL,), jnp.float32),
        scratch_types=[pltpu.VMEM((_CHUNK,), jnp.float32)],
        mesh=plsc.VectorSubcoreMesh(core_axis_name="c", subcore_axis_name="s"),
    )
    return run(xflat)


def kernel(x):
    B, C, H, W = x.shape
    out = _sc_kernel(x.reshape(B * C * H * W))
    return out.reshape(B, C, H, W)
